# baseline (speedup 1.0000x reference)
import functools

import jax
import jax.numpy as jnp
from jax import lax
from jax.experimental import pallas as pl
from jax.experimental.pallas import tpu as pltpu
from jax.experimental.pallas import tpu_sc as plsc

_B, _S, _T, _M = 16, 512, 2048, 80
_NPH = _B * _S
_NW = 32
_CHUNK = _NPH // _NW
_LANES = 16
_TBL = 32


def _sc_body(pitch_p, pitch_t, energy_p, energy_t, logdur_p, dur_t, mask_f,
             table, out, pp_v, pt_v, ep_v, et_v, lp_v, dt_v, m_v, tab_v,
             acc_v):
    wid = lax.axis_index("s") * 2 + lax.axis_index("c")
    base = wid * _CHUNK
    sl_in = pl.ds(base, _CHUNK)
    pltpu.sync_copy(pitch_p.at[sl_in], pp_v)
    pltpu.sync_copy(pitch_t.at[sl_in], pt_v)
    pltpu.sync_copy(energy_p.at[sl_in], ep_v)
    pltpu.sync_copy(energy_t.at[sl_in], et_v)
    pltpu.sync_copy(logdur_p.at[sl_in], lp_v)
    pltpu.sync_copy(dur_t.at[sl_in], dt_v)
    pltpu.sync_copy(mask_f.at[sl_in], m_v)
    pltpu.sync_copy(table, tab_v)

    acc_p = jnp.zeros((_LANES,), jnp.float32)
    acc_e = jnp.zeros((_LANES,), jnp.float32)
    acc_d = jnp.zeros((_LANES,), jnp.float32)
    acc_m = jnp.zeros((_LANES,), jnp.float32)
    for i in range(_CHUNK // _LANES):
        sl = pl.ds(i * _LANES, _LANES)
        m = 1.0 - m_v[sl]
        dp = pp_v[sl] - pt_v[sl]
        acc_p = acc_p + dp * dp * m
        de = ep_v[sl] - et_v[sl]
        acc_e = acc_e + de * de * m
        idx = jnp.minimum(jnp.maximum(dt_v[sl], 0), _TBL - 1)
        ld = plsc.load_gather(tab_v, [idx])
        dd = lp_v[sl] - ld
        acc_d = acc_d + dd * dd * m
        acc_m = acc_m + m

    acc_v[0, :] = acc_p
    acc_v[1, :] = acc_e
    acc_v[2, :] = acc_d
    acc_v[3, :] = acc_m
    pltpu.sync_copy(acc_v, out.at[wid])


def _sc_phoneme_losses(pitch_p, pitch_t, energy_p, energy_t, logdur_p, dur_t,
                       mask_f, table):
    mesh = plsc.VectorSubcoreMesh(core_axis_name="c", subcore_axis_name="s",
                                  num_cores=2, num_subcores=16)
    f = pl.kernel(
        _sc_body,
        out_type=jax.ShapeDtypeStruct((_NW, 4, _LANES), jnp.float32),
        mesh=mesh,
        compiler_params=pltpu.CompilerParams(needs_layout_passes=False),
        scratch_types=[
            pltpu.VMEM((_CHUNK,), jnp.float32),
            pltpu.VMEM((_CHUNK,), jnp.float32),
            pltpu.VMEM((_CHUNK,), jnp.float32),
            pltpu.VMEM((_CHUNK,), jnp.float32),
            pltpu.VMEM((_CHUNK,), jnp.float32),
            pltpu.VMEM((_CHUNK,), jnp.int32),
            pltpu.VMEM((_CHUNK,), jnp.float32),
            pltpu.VMEM((_TBL,), jnp.float32),
            pltpu.VMEM((4, _LANES), jnp.float32),
        ],
    )
    return f(pitch_p, pitch_t, energy_p, energy_t, logdur_p, dur_t, mask_f,
             table)


_LW = 640
_GR = _LW // _M
_ROWS = _B * _T * _M // _LW
_BR = 512


def _tc_body(melt_ref, melp_ref, pn_ref, mm_ref, out_ref):
    b = pl.program_id(0)
    lane = jax.lax.broadcasted_iota(jnp.int32, (_GR, _LW), 1)
    seg = jax.lax.broadcasted_iota(jnp.int32, (_GR, _LW), 0)
    e = (lane // _M == seg).astype(jnp.float32)
    mexp = jax.lax.dot_general(
        1.0 - mm_ref[...], e, (((1,), (0,)), ((), ())),
        preferred_element_type=jnp.float32)
    t = melt_ref[...]
    s_mel = jnp.sum(jnp.abs(melp_ref[...] - t) * mexp)
    s_pn = jnp.sum(jnp.abs(pn_ref[...] - t) * mexp)
    s_m = jnp.sum(1.0 - mm_ref[...])

    @pl.when(b == 0)
    def _init():
        out_ref[0] = s_mel
        out_ref[1] = s_pn
        out_ref[2] = s_m

    @pl.when(b != 0)
    def _acc():
        out_ref[0] += s_mel
        out_ref[1] += s_pn
        out_ref[2] += s_m


def _tc_mel_losses(mel_t, mel_p, pn_p, mel_mask_f):
    return pl.pallas_call(
        _tc_body,
        grid=(_ROWS // _BR,),
        in_specs=[
            pl.BlockSpec((_BR, _LW), lambda i: (i, 0)),
            pl.BlockSpec((_BR, _LW), lambda i: (i, 0)),
            pl.BlockSpec((_BR, _LW), lambda i: (i, 0)),
            pl.BlockSpec((_BR, _GR), lambda i: (i, 0)),
        ],
        out_specs=pl.BlockSpec(memory_space=pltpu.SMEM),
        out_shape=jax.ShapeDtypeStruct((3,), jnp.float32),
    )(mel_t, mel_p, pn_p, mel_mask_f)


def kernel(mel_targets, pitch_targets, energy_targets, duration_targets,
           mel_predictions, postnet_mel_predictions, pitch_predictions,
           energy_predictions, log_duration_predictions, src_masks,
           mel_masks):
    src_mask_f = src_masks.astype(jnp.float32).reshape(-1)
    mel_mask_f = mel_masks.astype(jnp.float32).reshape(_ROWS, _GR)
    dur_i = duration_targets.astype(jnp.int32).reshape(-1)
    table = jnp.log(jnp.arange(_TBL, dtype=jnp.float32) + 1.0)

    sc_part = _sc_phoneme_losses(
        pitch_predictions.reshape(-1), pitch_targets.reshape(-1),
        energy_predictions.reshape(-1), energy_targets.reshape(-1),
        log_duration_predictions.reshape(-1), dur_i, src_mask_f, table)
    tc_part = _tc_mel_losses(mel_targets.reshape(_ROWS, _LW),
                             mel_predictions.reshape(_ROWS, _LW),
                             postnet_mel_predictions.reshape(_ROWS, _LW),
                             mel_mask_f)

    sc_sums = jnp.sum(sc_part, axis=(0, 2))
    pitch_sq, energy_sq, dur_sq, src_cnt = (sc_sums[0], sc_sums[1],
                                            sc_sums[2], sc_sums[3])
    mel_abs, pn_abs, mel_cnt_rows = tc_part[0], tc_part[1], tc_part[2]

    src_den = jnp.maximum(src_cnt, 1.0)
    mel_den = jnp.maximum(mel_cnt_rows * _M, 1.0)
    pitch_loss = pitch_sq / src_den
    energy_loss = energy_sq / src_den
    duration_loss = dur_sq / src_den
    mel_loss = mel_abs / mel_den
    postnet_mel_loss = pn_abs / mel_den
    total_loss = (mel_loss + postnet_mel_loss + duration_loss + pitch_loss
                  + energy_loss)
    return (total_loss, mel_loss, postnet_mel_loss, pitch_loss, energy_loss,
            duration_loss)


# speedup vs baseline: 1.1907x; 1.1907x over previous
import functools

import jax
import jax.numpy as jnp
from jax import lax
from jax.experimental import pallas as pl
from jax.experimental.pallas import tpu as pltpu
from jax.experimental.pallas import tpu_sc as plsc

_B, _S, _T, _M = 16, 512, 2048, 80
_NPH = _B * _S
_NW = 32
_CHUNK = _NPH // _NW
_LANES = 16
_TBL = 32


def _sc_body(pitch_p, pitch_t, energy_p, energy_t, logdur_p, dur_t, mask_f,
             table, out, pp_v, pt_v, ep_v, et_v, lp_v, dt_v, m_v, tab_v,
             acc_v):
    wid = lax.axis_index("s") * 2 + lax.axis_index("c")
    base = wid * _CHUNK
    sl_in = pl.ds(base, _CHUNK)
    pltpu.sync_copy(pitch_p.at[sl_in], pp_v)
    pltpu.sync_copy(pitch_t.at[sl_in], pt_v)
    pltpu.sync_copy(energy_p.at[sl_in], ep_v)
    pltpu.sync_copy(energy_t.at[sl_in], et_v)
    pltpu.sync_copy(logdur_p.at[sl_in], lp_v)
    pltpu.sync_copy(dur_t.at[sl_in], dt_v)
    pltpu.sync_copy(mask_f.at[sl_in], m_v)
    pltpu.sync_copy(table, tab_v)

    acc_p = jnp.zeros((_LANES,), jnp.float32)
    acc_e = jnp.zeros((_LANES,), jnp.float32)
    acc_d = jnp.zeros((_LANES,), jnp.float32)
    acc_m = jnp.zeros((_LANES,), jnp.float32)
    for i in range(_CHUNK // _LANES):
        sl = pl.ds(i * _LANES, _LANES)
        m = 1.0 - m_v[sl]
        dp = pp_v[sl] - pt_v[sl]
        acc_p = acc_p + dp * dp * m
        de = ep_v[sl] - et_v[sl]
        acc_e = acc_e + de * de * m
        idx = jnp.minimum(jnp.maximum(dt_v[sl], 0), _TBL - 1)
        ld = plsc.load_gather(tab_v, [idx])
        dd = lp_v[sl] - ld
        acc_d = acc_d + dd * dd * m
        acc_m = acc_m + m

    acc_v[0, :] = acc_p
    acc_v[1, :] = acc_e
    acc_v[2, :] = acc_d
    acc_v[3, :] = acc_m
    pltpu.sync_copy(acc_v, out.at[wid])


def _sc_phoneme_losses(pitch_p, pitch_t, energy_p, energy_t, logdur_p, dur_t,
                       mask_f, table):
    mesh = plsc.VectorSubcoreMesh(core_axis_name="c", subcore_axis_name="s",
                                  num_cores=2, num_subcores=16)
    f = pl.kernel(
        _sc_body,
        out_type=jax.ShapeDtypeStruct((_NW, 4, _LANES), jnp.float32),
        mesh=mesh,
        compiler_params=pltpu.CompilerParams(needs_layout_passes=False),
        scratch_types=[
            pltpu.VMEM((_CHUNK,), jnp.float32),
            pltpu.VMEM((_CHUNK,), jnp.float32),
            pltpu.VMEM((_CHUNK,), jnp.float32),
            pltpu.VMEM((_CHUNK,), jnp.float32),
            pltpu.VMEM((_CHUNK,), jnp.float32),
            pltpu.VMEM((_CHUNK,), jnp.int32),
            pltpu.VMEM((_CHUNK,), jnp.float32),
            pltpu.VMEM((_TBL,), jnp.float32),
            pltpu.VMEM((4, _LANES), jnp.float32),
        ],
    )
    return f(pitch_p, pitch_t, energy_p, energy_t, logdur_p, dur_t, mask_f,
             table)


_BR = 4096


def _tc_body(melt_ref, melp_ref, pn_ref, mm_ref, out_ref):
    b = pl.program_id(0)
    mexp = 1.0 - mm_ref[...]
    t = melt_ref[...]
    s_mel = jnp.sum(jnp.abs(melp_ref[...] - t) * mexp)
    s_pn = jnp.sum(jnp.abs(pn_ref[...] - t) * mexp)
    s_m = jnp.sum(1.0 - mm_ref[...])

    @pl.when(b == 0)
    def _init():
        out_ref[0] = s_mel
        out_ref[1] = s_pn
        out_ref[2] = s_m

    @pl.when(b != 0)
    def _acc():
        out_ref[0] += s_mel
        out_ref[1] += s_pn
        out_ref[2] += s_m


def _tc_mel_losses(mel_t, mel_p, pn_p, mel_mask_f):
    rows = _B * _T
    return pl.pallas_call(
        _tc_body,
        grid=(rows // _BR,),
        in_specs=[
            pl.BlockSpec((_BR, _M), lambda i: (i, 0)),
            pl.BlockSpec((_BR, _M), lambda i: (i, 0)),
            pl.BlockSpec((_BR, _M), lambda i: (i, 0)),
            pl.BlockSpec((_BR, 1), lambda i: (i, 0)),
        ],
        out_specs=pl.BlockSpec(memory_space=pltpu.SMEM),
        out_shape=jax.ShapeDtypeStruct((3,), jnp.float32),
    )(mel_t, mel_p, pn_p, mel_mask_f)


def kernel(mel_targets, pitch_targets, energy_targets, duration_targets,
           mel_predictions, postnet_mel_predictions, pitch_predictions,
           energy_predictions, log_duration_predictions, src_masks,
           mel_masks):
    src_mask_f = src_masks.astype(jnp.float32).reshape(-1)
    mel_mask_f = mel_masks.astype(jnp.float32).reshape(_B * _T, 1)
    dur_i = duration_targets.astype(jnp.int32).reshape(-1)
    table = jnp.log(jnp.arange(_TBL, dtype=jnp.float32) + 1.0)

    sc_part = _sc_phoneme_losses(
        pitch_predictions.reshape(-1), pitch_targets.reshape(-1),
        energy_predictions.reshape(-1), energy_targets.reshape(-1),
        log_duration_predictions.reshape(-1), dur_i, src_mask_f, table)
    tc_part = _tc_mel_losses(mel_targets.reshape(_B * _T, _M),
                             mel_predictions.reshape(_B * _T, _M),
                             postnet_mel_predictions.reshape(_B * _T, _M),
                             mel_mask_f)

    sc_sums = jnp.sum(sc_part, axis=(0, 2))
    pitch_sq, energy_sq, dur_sq, src_cnt = (sc_sums[0], sc_sums[1],
                                            sc_sums[2], sc_sums[3])
    mel_abs, pn_abs, mel_cnt_rows = tc_part[0], tc_part[1], tc_part[2]

    src_den = jnp.maximum(src_cnt, 1.0)
    mel_den = jnp.maximum(mel_cnt_rows * _M, 1.0)
    pitch_loss = pitch_sq / src_den
    energy_loss = energy_sq / src_den
    duration_loss = dur_sq / src_den
    mel_loss = mel_abs / mel_den
    postnet_mel_loss = pn_abs / mel_den
    total_loss = (mel_loss + postnet_mel_loss + duration_loss + pitch_loss
                  + energy_loss)
    return (total_loss, mel_loss, postnet_mel_loss, pitch_loss, energy_loss,
            duration_loss)
